# Initial kernel scaffold; baseline (speedup 1.0000x reference)
import jax
import jax.numpy as jnp
from jax.experimental import pallas as pl


def kernel(word_id, shape_id, word_table, shape_table):
    raise NotImplementedError("write your pallas kernel here")


# speedup vs baseline: 5.2000x; 5.2000x over previous
import functools

import jax
import jax.numpy as jnp
from jax import lax
from jax.experimental import pallas as pl
from jax.experimental.pallas import tpu as pltpu
from jax.experimental.pallas import tpu_sc as plsc

WORD_DIM = 128
SHAPE_DIM = 32
OUT_DIM = WORD_DIM + SHAPE_DIM
NUM_WORKERS = 32
NB = 4


def kernel(word_id, shape_id, word_table, shape_table):
    B, L = word_id.shape
    b_per_w = B // NUM_WORKERS
    steps = b_per_w // NB
    rows_per_chunk = NB * L
    groups = (rows_per_chunk + 15) // 16
    shape_vocab = shape_table.shape[0]

    mesh = plsc.VectorSubcoreMesh(core_axis_name="c", subcore_axis_name="s")

    @functools.partial(
        pl.kernel,
        mesh=mesh,
        out_type=jax.ShapeDtypeStruct((B, L, OUT_DIM), jnp.float32),
        scratch_types=[
            pltpu.VMEM((NB, L), jnp.int32),
            pltpu.VMEM((NB, L), jnp.int32),
            pltpu.VMEM((NB, L, OUT_DIM), jnp.float32),
            pltpu.VMEM((shape_vocab * SHAPE_DIM,), jnp.float32),
            pltpu.SemaphoreType.DMA,
        ],
    )
    def sc_kernel(wid_hbm, sid_hbm, wtab_hbm, stab_hbm, out_hbm,
                  widx_v, sidx_v, obuf_v, stab_v, sem):
        w = lax.axis_index("s") * 2 + lax.axis_index("c")
        b_start = w * b_per_w

        pltpu.sync_copy(stab_hbm, stab_v)

        lanes = lax.iota(jnp.int32, 16)

        def chunk_body(i, carry):
            b0 = b_start + i * NB
            pltpu.sync_copy(wid_hbm.at[pl.ds(b0, NB)], widx_v)
            pltpu.sync_copy(sid_hbm.at[pl.ds(b0, NB)], sidx_v)

            copies = []
            for s in range(NB):
                copies.append(pltpu.async_copy(
                    wtab_hbm.at[widx_v.at[s]],
                    obuf_v.at[s, :, pl.ds(0, WORD_DIM)],
                    sem))

            group_bases = list(range(0, L - 16, 16)) + [L - 16]
            for s in range(NB):
                for base_l in group_bases:
                    rows16 = sidx_v[s, pl.ds(base_l, 16)]
                    for k in range(16):
                        l = base_l + k
                        base = rows16[k] * SHAPE_DIM
                        for h in range(SHAPE_DIM // 16):
                            vals = stab_v[pl.ds(base + h * 16, 16)]
                            obuf_v[s, l, pl.ds(WORD_DIM + h * 16, 16)] = vals

            for c in copies:
                c.wait()

            pltpu.sync_copy(obuf_v, out_hbm.at[pl.ds(b0, NB)])
            return carry

        lax.fori_loop(0, steps, chunk_body, 0)

    out = sc_kernel(word_id, shape_id, word_table,
                    shape_table.reshape(shape_vocab * SHAPE_DIM))
    return out
